# baseline (speedup 1.0000x reference)
import functools

import jax
import jax.numpy as jnp
from jax import lax
from jax.experimental import pallas as pl
from jax.experimental.pallas import tpu as pltpu
from jax.experimental.pallas import tpu_sc as plsc

HID = 128
NSC = 2
NTILE = 16
CHUNK = 10112
NDUMP = 8
GB = 128
BIG = 1 << 30


def _pad1(x, n, val):
    return jnp.pad(x, (0, n - x.shape[0]), constant_values=val)


IC = 3840


def _make_scatter(n_pairs_pad, n_dest):
    assert n_pairs_pad % (NTILE * IC) == 0
    ppt = n_pairs_pad // NTILE
    nchunks = ppt // IC
    npass = -(-n_dest // (NSC * CHUNK))
    n_out_pad = npass * NSC * CHUNK
    zrows = CHUNK // NTILE
    mesh = plsc.VectorSubcoreMesh(core_axis_name="c", subcore_axis_name="s")

    @functools.partial(
        pl.kernel,
        out_type=jax.ShapeDtypeStruct((n_out_pad, HID), jnp.float32),
        mesh=mesh,
        scratch_types=[
            pltpu.VMEM((IC,), jnp.int32),
            pltpu.VMEM((IC,), jnp.int32),
            pltpu.VMEM((IC + 2 * GB,), jnp.int32),
            pltpu.VMEM((IC + 2 * GB,), jnp.int32),
            pltpu.VMEM((GB,), jnp.int32),
            pltpu.VMEM((GB, HID), jnp.float32),
            pltpu.VMEM((GB, HID), jnp.float32),
            pltpu.VMEM_SHARED((CHUNK + NDUMP, HID), jnp.float32),
            pltpu.SemaphoreType.DMA,
            pltpu.SemaphoreType.DMA,
        ],
        compiler_params=pltpu.CompilerParams(needs_layout_passes=False),
    )
    def k(table, srcs, dsts, zeros_h, out,
          srcv, dstv, msrc, mdst, idxb, rows0, rows1, shared, sem0, sem1):
        c = lax.axis_index("c")
        s = lax.axis_index("s")

        def one_pass(p, _):
            base = (p * NSC + c) * CHUNK
            for z in range(zrows // GB):
                pltpu.sync_copy(zeros_h,
                                shared.at[pl.ds(s * zrows + z * GB, GB)])
            if zrows % GB:
                rem = zrows % GB
                pltpu.sync_copy(
                    zeros_h.at[pl.ds(0, rem)],
                    shared.at[pl.ds(s * zrows + (zrows // GB) * GB, rem)])
            plsc.subcore_barrier()

            def chunk_body(kk, _):
                pltpu.sync_copy(srcs.at[pl.ds(s * ppt + kk * IC, IC)], srcv)
                pltpu.sync_copy(dsts.at[pl.ds(s * ppt + kk * IC, IC)], dstv)

                def scan(i, cnt):
                    dv = dstv[pl.ds(i * 16, 16)]
                    sv = srcv[pl.ds(i * 16, 16)]
                    lo = dv - base
                    m = (lo >= 0) & (lo < CHUNK)
                    mi = jnp.where(m, 1, 0)
                    csum = plsc.cumsum(mi)
                    pos = (cnt - 1) + csum
                    plsc.store_scatter(msrc, [pos], sv, mask=m)
                    plsc.store_scatter(mdst, [pos], lo, mask=m)
                    return cnt + jnp.squeeze(lax.slice(csum, (15,), (16,)))

                cnt = lax.fori_loop(0, IC // 16, scan, jnp.int32(0),
                                    unroll=4)
                for t in range(GB // 16):
                    lane = lax.iota(jnp.int32, 16)
                    msrc[pl.ds(cnt + t * 16, 16)] = (lane + s * 16) % 64
                    mdst[pl.ds(cnt + t * 16, 16)] = CHUNK + (lane + t) % NDUMP
                nb = (cnt + GB - 1) // GB

                @pl.when(nb > 0)
                def _():
                    pltpu.async_copy(
                        table.at[msrc.at[pl.ds(0, GB)]], rows0, sem0)

                def proc2(j2, _):
                    for par in range(2):
                        rbuf, rsem = (rows0, sem0) if par == 0 else (rows1, sem1)
                        obuf, osem = (rows1, sem1) if par == 0 else (rows0, sem0)
                        j = j2 * 2 + par

                        @pl.when(j < nb)
                        def _(j=j, rbuf=rbuf, rsem=rsem, obuf=obuf, osem=osem):
                            pltpu.make_async_copy(
                                table.at[msrc.at[pl.ds(j * GB, GB)]],
                                rbuf, rsem).wait()

                            @pl.when(j + 1 < nb)
                            def _():
                                pltpu.async_copy(
                                    table.at[msrc.at[pl.ds((j + 1) * GB, GB)]],
                                    obuf, osem)

                            for q in range(GB // 16):
                                idxb[pl.ds(q * 16, 16)] = (
                                    mdst[pl.ds(j * GB + q * 16, 16)])
                            pltpu.sync_copy(rbuf, shared.at[idxb], add=True)
                    return 0

                lax.fori_loop(0, (nb + 1) // 2, proc2, 0)
                return 0

            lax.fori_loop(0, nchunks, chunk_body, 0)
            plsc.subcore_barrier()
            pltpu.sync_copy(shared.at[pl.ds(s * zrows, zrows)],
                            out.at[pl.ds(base + s * zrows, zrows)])
            return 0

        lax.fori_loop(0, npass, one_pass, 0)

    return k


def _make_gather(n_rows_pad):
    assert n_rows_pad % (NSC * NTILE * GB) == 0
    per_w = n_rows_pad // (NSC * NTILE)
    nb = per_w // GB
    mesh = plsc.VectorSubcoreMesh(core_axis_name="c", subcore_axis_name="s")

    @functools.partial(
        pl.kernel,
        out_type=jax.ShapeDtypeStruct((n_rows_pad, HID), jnp.float32),
        mesh=mesh,
        scratch_types=[
            pltpu.VMEM((per_w,), jnp.int32),
            pltpu.VMEM((GB, HID), jnp.float32),
            pltpu.SemaphoreType.DMA,
        ],
        compiler_params=pltpu.CompilerParams(needs_layout_passes=False),
    )
    def k(table, idx, out, idxv, rows, sem):
        c = lax.axis_index("c")
        s = lax.axis_index("s")
        base = (s * NSC + c) * per_w
        pltpu.sync_copy(idx.at[pl.ds(base, per_w)], idxv)

        def body(j, _):
            pltpu.async_copy(table.at[idxv.at[pl.ds(j * GB, GB)]], rows, sem
                             ).wait()
            pltpu.sync_copy(rows, out.at[pl.ds(base + j * GB, GB)])
            return 0

        lax.fori_loop(0, nb, body, 0)

    return k


_BLK = 1024


def _mm_plain(x, w, n_valid):
    cout = w.shape[1]
    nb = pl.cdiv(n_valid, _BLK)

    def body(x_ref, w_ref, out_ref):
        out_ref[...] = jnp.dot(x_ref[...], w_ref[...],
                               preferred_element_type=jnp.float32)

    return pl.pallas_call(
        body,
        grid=(nb,),
        in_specs=[pl.BlockSpec((_BLK, x.shape[1]), lambda i: (i, 0)),
                  pl.BlockSpec(w.shape, lambda i: (0, 0))],
        out_specs=pl.BlockSpec((_BLK, cout), lambda i: (i, 0)),
        out_shape=jax.ShapeDtypeStruct((nb * _BLK, cout), jnp.float32),
    )(x, w)


def _mm_stats(parts, w, n_valid, pre=None):
    npart = len(parts)
    cout = w.shape[1]
    nb = pl.cdiv(n_valid, _BLK)
    has_pre = pre is not None
    ins = list(parts) + ([pre] if has_pre else [])

    def body(*refs):
        part_refs = refs[:npart]
        pre_ref = refs[npart] if has_pre else None
        w_ref, out_ref, st_ref = refs[npart + has_pre:]
        i = pl.program_id(0)
        rowid = lax.broadcasted_iota(jnp.int32, (_BLK, 1), 0) + i * _BLK
        valid = rowid < n_valid
        h = pre_ref[...] if has_pre else jnp.zeros((_BLK, cout), jnp.float32)
        for kk in range(npart):
            h = h + jnp.dot(part_refs[kk][...],
                            w_ref[kk * HID:(kk + 1) * HID, :],
                            preferred_element_type=jnp.float32)
        h = jnp.where(valid, h, 0.0)
        out_ref[...] = h
        su = jnp.sum(h, axis=0, keepdims=True)
        sq = jnp.sum(h * h, axis=0, keepdims=True)
        upd = jnp.concatenate([su, sq, jnp.zeros((6, cout), jnp.float32)], 0)

        @pl.when(i == 0)
        def _():
            st_ref[...] = upd

        @pl.when(i > 0)
        def _():
            st_ref[...] += upd

    out, st = pl.pallas_call(
        body,
        grid=(nb,),
        in_specs=[pl.BlockSpec((_BLK, HID), lambda i: (i, 0))] * npart
        + ([pl.BlockSpec((_BLK, cout), lambda i: (i, 0))] if has_pre else [])
        + [pl.BlockSpec(w.shape, lambda i: (0, 0))],
        out_specs=[pl.BlockSpec((_BLK, cout), lambda i: (i, 0)),
                   pl.BlockSpec((8, cout), lambda i: (0, 0))],
        out_shape=[jax.ShapeDtypeStruct((nb * _BLK, cout), jnp.float32),
                   jax.ShapeDtypeStruct((8, cout), jnp.float32)],
    )(*ins, w)
    return out, st


def _bn_mm_stats(x, ss, w, n_valid):
    cout = w.shape[1]
    nb = x.shape[0] // _BLK

    def body(x_ref, ss_ref, w_ref, out_ref, st_ref):
        i = pl.program_id(0)
        rowid = lax.broadcasted_iota(jnp.int32, (_BLK, 1), 0) + i * _BLK
        valid = rowid < n_valid
        a = jnp.maximum(x_ref[...] * ss_ref[0:1, :] + ss_ref[1:2, :], 0.0)
        a = jnp.where(valid, a, 0.0)
        h = jnp.dot(a, w_ref[...], preferred_element_type=jnp.float32)
        out_ref[...] = h
        su = jnp.sum(h, axis=0, keepdims=True)
        sq = jnp.sum(h * h, axis=0, keepdims=True)
        upd = jnp.concatenate([su, sq, jnp.zeros((6, cout), jnp.float32)], 0)

        @pl.when(i == 0)
        def _():
            st_ref[...] = upd

        @pl.when(i > 0)
        def _():
            st_ref[...] += upd

    return pl.pallas_call(
        body,
        grid=(nb,),
        in_specs=[pl.BlockSpec((_BLK, x.shape[1]), lambda i: (i, 0)),
                  pl.BlockSpec(ss.shape, lambda i: (0, 0)),
                  pl.BlockSpec(w.shape, lambda i: (0, 0))],
        out_specs=[pl.BlockSpec((_BLK, cout), lambda i: (i, 0)),
                   pl.BlockSpec((8, cout), lambda i: (0, 0))],
        out_shape=[jax.ShapeDtypeStruct((nb * _BLK, cout), jnp.float32),
                   jax.ShapeDtypeStruct((8, cout), jnp.float32)],
    )(x, ss, w)


def _bn_apply(x, ss, n_valid):
    c = x.shape[1]
    blk = 1024
    nb = pl.cdiv(n_valid, blk)

    def body(x_ref, ss_ref, out_ref):
        out_ref[...] = jnp.maximum(
            x_ref[...] * ss_ref[0:1, :] + ss_ref[1:2, :], 0.0)

    return pl.pallas_call(
        body,
        grid=(nb,),
        in_specs=[pl.BlockSpec((blk, c), lambda i: (i, 0)),
                  pl.BlockSpec(ss.shape, lambda i: (0, 0))],
        out_specs=pl.BlockSpec((blk, c), lambda i: (i, 0)),
        out_shape=jax.ShapeDtypeStruct((n_valid, c), jnp.float32),
    )(x, ss)


def _scale_shift(st, g, b, n):
    mu = st[0] / n
    var = st[1] / n - mu * mu
    scale = g * lax.rsqrt(var + 1e-5)
    shift = b - mu * scale
    return jnp.concatenate(
        [scale[None], shift[None], jnp.zeros((6, scale.shape[0]), jnp.float32)], 0)


def kernel(edge_rep, cycle_rep, e2c_src, e2c_dst, cycle_id, c2e_src, c2e_dst,
           cyc_W1, cyc_g1, cyc_b1, cyc_W2, cyc_g2, cyc_b2,
           edg_W1, edg_g1, edg_b1, edg_W2, edg_g2, edg_b2):
    E, _ = edge_rep.shape
    C_ROWS, _ = cycle_rep.shape
    P1 = e2c_src.shape[0]
    P2 = c2e_src.shape[0]
    N_CYC = 30000

    zeros_h = jnp.zeros((GB, HID), jnp.float32)

    def rup(n, m):
        return (n + m - 1) // m * m

    ca = _mm_plain(cycle_rep, cyc_W1[0:HID, :], C_ROWS)
    ea = _mm_plain(edge_rep, edg_W1[0:HID, :], E)

    p1p = rup(P1, NTILE * IC)
    e2c = _make_scatter(p1p, C_ROWS)(
        edge_rep,
        _pad1(e2c_src, p1p, 0),
        _pad1(e2c_dst, p1p, BIG),
        zeros_h)

    cp = rup(C_ROWS, NTILE * IC)
    cyc_sum = _make_scatter(cp, N_CYC)(
        e2c,
        _pad1(jnp.arange(C_ROWS, dtype=jnp.int32), cp, 0),
        _pad1(cycle_id.astype(jnp.int32), cp, BIG),
        zeros_h)

    gp = rup(C_ROWS, NSC * NTILE * GB)
    cyc_gath = _make_gather(gp)(
        cyc_sum, _pad1(cycle_id.astype(jnp.int32), gp, 0))

    h1, st1 = _mm_stats([cyc_gath, e2c], cyc_W1[HID:, :], C_ROWS, pre=ca)
    ss1 = _scale_shift(st1, cyc_g1, cyc_b1, C_ROWS)
    h2, st2 = _bn_mm_stats(h1, ss1, cyc_W2, C_ROWS)
    ss2 = _scale_shift(st2, cyc_g2, cyc_b2, C_ROWS)
    cycle_out = _bn_apply(h2, ss2, C_ROWS)

    p2p = rup(P2, NTILE * IC)
    c2e = _make_scatter(p2p, E)(
        cycle_out,
        _pad1(c2e_src, p2p, 0),
        _pad1(c2e_dst, p2p, BIG),
        zeros_h)

    g1h, st3 = _mm_stats([c2e], edg_W1[HID:, :], E, pre=ea)
    ss3 = _scale_shift(st3, edg_g1, edg_b1, E)
    g2h, st4 = _bn_mm_stats(g1h, ss3, edg_W2, E)
    ss4 = _scale_shift(st4, edg_g2, edg_b2, E)
    edge_out = _bn_apply(g2h, ss4, E)

    return (edge_out, cycle_out)


# speedup vs baseline: 1.0490x; 1.0490x over previous
import functools

import jax
import jax.numpy as jnp
from jax import lax
from jax.experimental import pallas as pl
from jax.experimental.pallas import tpu as pltpu
from jax.experimental.pallas import tpu_sc as plsc

HID = 128
NSC = 2
NTILE = 16
CHUNK = 10112
NDUMP = 8
GB = 128
BIG = 1 << 30


def _pad1(x, n, val):
    return jnp.pad(x, (0, n - x.shape[0]), constant_values=val)


IC = 3840


def _make_scatter(n_pairs_pad, n_dest):
    assert n_pairs_pad % (NTILE * IC) == 0
    ppt = n_pairs_pad // NTILE
    nchunks = ppt // IC
    npass = -(-n_dest // (NSC * CHUNK))
    n_out_pad = npass * NSC * CHUNK
    zrows = CHUNK // NTILE
    mesh = plsc.VectorSubcoreMesh(core_axis_name="c", subcore_axis_name="s")

    @functools.partial(
        pl.kernel,
        out_type=jax.ShapeDtypeStruct((n_out_pad, HID), jnp.float32),
        mesh=mesh,
        scratch_types=[
            pltpu.VMEM((IC,), jnp.int32),
            pltpu.VMEM((IC,), jnp.int32),
            pltpu.VMEM((IC + 2 * GB,), jnp.int32),
            pltpu.VMEM((IC + 2 * GB,), jnp.int32),
            pltpu.VMEM((GB,), jnp.int32),
            pltpu.VMEM((GB, HID), jnp.float32),
            pltpu.VMEM((GB, HID), jnp.float32),
            pltpu.VMEM_SHARED((CHUNK + NDUMP, HID), jnp.float32),
            pltpu.SemaphoreType.DMA,
            pltpu.SemaphoreType.DMA,
        ],
        compiler_params=pltpu.CompilerParams(needs_layout_passes=False),
    )
    def k(table, srcs, dsts, zeros_h, out,
          srcv, dstv, msrc, mdst, idxb, rows0, rows1, shared, sem0, sem1):
        c = lax.axis_index("c")
        s = lax.axis_index("s")

        def one_pass(p, _):
            base = (p * NSC + c) * CHUNK
            for z in range(zrows // GB):
                pltpu.sync_copy(zeros_h,
                                shared.at[pl.ds(s * zrows + z * GB, GB)])
            if zrows % GB:
                rem = zrows % GB
                pltpu.sync_copy(
                    zeros_h.at[pl.ds(0, rem)],
                    shared.at[pl.ds(s * zrows + (zrows // GB) * GB, rem)])
            plsc.subcore_barrier()

            def chunk_body(kk, _):
                pltpu.sync_copy(srcs.at[pl.ds(s * ppt + kk * IC, IC)], srcv)
                pltpu.sync_copy(dsts.at[pl.ds(s * ppt + kk * IC, IC)], dstv)

                def scan(i, cnt):
                    dv = dstv[pl.ds(i * 16, 16)]
                    sv = srcv[pl.ds(i * 16, 16)]
                    lo = dv - base
                    m = (lo >= 0) & (lo < CHUNK)
                    mi = jnp.where(m, 1, 0)
                    csum = plsc.cumsum(mi)
                    pos = (cnt - 1) + csum
                    plsc.store_scatter(msrc, [pos], sv, mask=m)
                    plsc.store_scatter(mdst, [pos], lo, mask=m)
                    return cnt + jnp.squeeze(lax.slice(csum, (15,), (16,)))

                cnt = lax.fori_loop(0, IC // 16, scan, jnp.int32(0),
                                    unroll=4)
                for t in range(GB // 16):
                    lane = lax.iota(jnp.int32, 16)
                    msrc[pl.ds(cnt + t * 16, 16)] = (lane + s * 16) % 64
                    mdst[pl.ds(cnt + t * 16, 16)] = CHUNK + (lane + t) % NDUMP
                nb = (cnt + GB - 1) // GB

                @pl.when(nb > 0)
                def _():
                    pltpu.async_copy(
                        table.at[msrc.at[pl.ds(0, GB)]], rows0, sem0)

                def proc2(j2, _):
                    for par in range(2):
                        rbuf, rsem = (rows0, sem0) if par == 0 else (rows1, sem1)
                        obuf, osem = (rows1, sem1) if par == 0 else (rows0, sem0)
                        j = j2 * 2 + par

                        @pl.when(j < nb)
                        def _(j=j, rbuf=rbuf, rsem=rsem, obuf=obuf, osem=osem):
                            pltpu.make_async_copy(
                                table.at[msrc.at[pl.ds(j * GB, GB)]],
                                rbuf, rsem).wait()

                            @pl.when(j + 1 < nb)
                            def _():
                                pltpu.async_copy(
                                    table.at[msrc.at[pl.ds((j + 1) * GB, GB)]],
                                    obuf, osem)

                            for q in range(GB // 16):
                                idxb[pl.ds(q * 16, 16)] = (
                                    mdst[pl.ds(j * GB + q * 16, 16)])
                            pltpu.sync_copy(rbuf, shared.at[idxb], add=True)
                    return 0

                lax.fori_loop(0, (nb + 1) // 2, proc2, 0)
                return 0

            lax.fori_loop(0, nchunks, chunk_body, 0)
            plsc.subcore_barrier()
            pltpu.sync_copy(shared.at[pl.ds(s * zrows, zrows)],
                            out.at[pl.ds(base + s * zrows, zrows)])
            return 0

        lax.fori_loop(0, npass, one_pass, 0)

    return k


def _make_gather(n_rows_pad):
    assert n_rows_pad % (NSC * NTILE * GB) == 0
    per_w = n_rows_pad // (NSC * NTILE)
    nb = per_w // GB
    mesh = plsc.VectorSubcoreMesh(core_axis_name="c", subcore_axis_name="s")

    @functools.partial(
        pl.kernel,
        out_type=jax.ShapeDtypeStruct((n_rows_pad, HID), jnp.float32),
        mesh=mesh,
        scratch_types=[
            pltpu.VMEM((per_w,), jnp.int32),
            pltpu.VMEM((GB, HID), jnp.float32),
            pltpu.VMEM((GB, HID), jnp.float32),
            pltpu.SemaphoreType.DMA,
            pltpu.SemaphoreType.DMA,
        ],
        compiler_params=pltpu.CompilerParams(needs_layout_passes=False),
    )
    def k(table, idx, out, idxv, rows0, rows1, sem0, sem1):
        c = lax.axis_index("c")
        s = lax.axis_index("s")
        base = (s * NSC + c) * per_w
        pltpu.sync_copy(idx.at[pl.ds(base, per_w)], idxv)
        pltpu.async_copy(table.at[idxv.at[pl.ds(0, GB)]], rows0, sem0)

        def body(j2, _):
            for par in range(2):
                rbuf, rsem = (rows0, sem0) if par == 0 else (rows1, sem1)
                obuf, osem = (rows1, sem1) if par == 0 else (rows0, sem0)
                j = j2 * 2 + par

                @pl.when(j < nb)
                def _(j=j, rbuf=rbuf, rsem=rsem, obuf=obuf, osem=osem):
                    pltpu.make_async_copy(
                        table.at[idxv.at[pl.ds(j * GB, GB)]], rbuf, rsem
                    ).wait()

                    @pl.when(j + 1 < nb)
                    def _():
                        pltpu.async_copy(
                            table.at[idxv.at[pl.ds((j + 1) * GB, GB)]],
                            obuf, osem)

                    pltpu.sync_copy(rbuf, out.at[pl.ds(base + j * GB, GB)])
            return 0

        lax.fori_loop(0, (nb + 1) // 2, body, 0)

    return k


_BLK = 1024


def _mm_plain(x, w, n_valid):
    cout = w.shape[1]
    nb = pl.cdiv(n_valid, _BLK)

    def body(x_ref, w_ref, out_ref):
        out_ref[...] = jnp.dot(x_ref[...], w_ref[...],
                               preferred_element_type=jnp.float32)

    return pl.pallas_call(
        body,
        grid=(nb,),
        in_specs=[pl.BlockSpec((_BLK, x.shape[1]), lambda i: (i, 0)),
                  pl.BlockSpec(w.shape, lambda i: (0, 0))],
        out_specs=pl.BlockSpec((_BLK, cout), lambda i: (i, 0)),
        out_shape=jax.ShapeDtypeStruct((nb * _BLK, cout), jnp.float32),
    )(x, w)


def _mm_stats(parts, w, n_valid, pre=None):
    npart = len(parts)
    cout = w.shape[1]
    nb = pl.cdiv(n_valid, _BLK)
    has_pre = pre is not None
    ins = list(parts) + ([pre] if has_pre else [])

    def body(*refs):
        part_refs = refs[:npart]
        pre_ref = refs[npart] if has_pre else None
        w_ref, out_ref, st_ref = refs[npart + has_pre:]
        i = pl.program_id(0)
        rowid = lax.broadcasted_iota(jnp.int32, (_BLK, 1), 0) + i * _BLK
        valid = rowid < n_valid
        h = pre_ref[...] if has_pre else jnp.zeros((_BLK, cout), jnp.float32)
        for kk in range(npart):
            h = h + jnp.dot(part_refs[kk][...],
                            w_ref[kk * HID:(kk + 1) * HID, :],
                            preferred_element_type=jnp.float32)
        h = jnp.where(valid, h, 0.0)
        out_ref[...] = h
        su = jnp.sum(h, axis=0, keepdims=True)
        sq = jnp.sum(h * h, axis=0, keepdims=True)
        upd = jnp.concatenate([su, sq, jnp.zeros((6, cout), jnp.float32)], 0)

        @pl.when(i == 0)
        def _():
            st_ref[...] = upd

        @pl.when(i > 0)
        def _():
            st_ref[...] += upd

    out, st = pl.pallas_call(
        body,
        grid=(nb,),
        in_specs=[pl.BlockSpec((_BLK, HID), lambda i: (i, 0))] * npart
        + ([pl.BlockSpec((_BLK, cout), lambda i: (i, 0))] if has_pre else [])
        + [pl.BlockSpec(w.shape, lambda i: (0, 0))],
        out_specs=[pl.BlockSpec((_BLK, cout), lambda i: (i, 0)),
                   pl.BlockSpec((8, cout), lambda i: (0, 0))],
        out_shape=[jax.ShapeDtypeStruct((nb * _BLK, cout), jnp.float32),
                   jax.ShapeDtypeStruct((8, cout), jnp.float32)],
    )(*ins, w)
    return out, st


def _bn_mm_stats(x, ss, w, n_valid):
    cout = w.shape[1]
    nb = x.shape[0] // _BLK

    def body(x_ref, ss_ref, w_ref, out_ref, st_ref):
        i = pl.program_id(0)
        rowid = lax.broadcasted_iota(jnp.int32, (_BLK, 1), 0) + i * _BLK
        valid = rowid < n_valid
        a = jnp.maximum(x_ref[...] * ss_ref[0:1, :] + ss_ref[1:2, :], 0.0)
        a = jnp.where(valid, a, 0.0)
        h = jnp.dot(a, w_ref[...], preferred_element_type=jnp.float32)
        out_ref[...] = h
        su = jnp.sum(h, axis=0, keepdims=True)
        sq = jnp.sum(h * h, axis=0, keepdims=True)
        upd = jnp.concatenate([su, sq, jnp.zeros((6, cout), jnp.float32)], 0)

        @pl.when(i == 0)
        def _():
            st_ref[...] = upd

        @pl.when(i > 0)
        def _():
            st_ref[...] += upd

    return pl.pallas_call(
        body,
        grid=(nb,),
        in_specs=[pl.BlockSpec((_BLK, x.shape[1]), lambda i: (i, 0)),
                  pl.BlockSpec(ss.shape, lambda i: (0, 0)),
                  pl.BlockSpec(w.shape, lambda i: (0, 0))],
        out_specs=[pl.BlockSpec((_BLK, cout), lambda i: (i, 0)),
                   pl.BlockSpec((8, cout), lambda i: (0, 0))],
        out_shape=[jax.ShapeDtypeStruct((nb * _BLK, cout), jnp.float32),
                   jax.ShapeDtypeStruct((8, cout), jnp.float32)],
    )(x, ss, w)


def _bn_apply(x, ss, n_valid):
    c = x.shape[1]
    blk = 1024
    nb = pl.cdiv(n_valid, blk)

    def body(x_ref, ss_ref, out_ref):
        out_ref[...] = jnp.maximum(
            x_ref[...] * ss_ref[0:1, :] + ss_ref[1:2, :], 0.0)

    return pl.pallas_call(
        body,
        grid=(nb,),
        in_specs=[pl.BlockSpec((blk, c), lambda i: (i, 0)),
                  pl.BlockSpec(ss.shape, lambda i: (0, 0))],
        out_specs=pl.BlockSpec((blk, c), lambda i: (i, 0)),
        out_shape=jax.ShapeDtypeStruct((n_valid, c), jnp.float32),
    )(x, ss)


def _scale_shift(st, g, b, n):
    mu = st[0] / n
    var = st[1] / n - mu * mu
    scale = g * lax.rsqrt(var + 1e-5)
    shift = b - mu * scale
    return jnp.concatenate(
        [scale[None], shift[None], jnp.zeros((6, scale.shape[0]), jnp.float32)], 0)


def kernel(edge_rep, cycle_rep, e2c_src, e2c_dst, cycle_id, c2e_src, c2e_dst,
           cyc_W1, cyc_g1, cyc_b1, cyc_W2, cyc_g2, cyc_b2,
           edg_W1, edg_g1, edg_b1, edg_W2, edg_g2, edg_b2):
    E, _ = edge_rep.shape
    C_ROWS, _ = cycle_rep.shape
    P1 = e2c_src.shape[0]
    P2 = c2e_src.shape[0]
    N_CYC = 30000

    zeros_h = jnp.zeros((GB, HID), jnp.float32)

    def rup(n, m):
        return (n + m - 1) // m * m

    p1p = rup(P1, NTILE * IC)
    e2c = _make_scatter(p1p, C_ROWS)(
        edge_rep,
        _pad1(e2c_src, p1p, 0),
        _pad1(e2c_dst, p1p, BIG),
        zeros_h)

    cp = rup(C_ROWS, NTILE * IC)
    cyc_sum = _make_scatter(cp, N_CYC)(
        e2c,
        _pad1(jnp.arange(C_ROWS, dtype=jnp.int32), cp, 0),
        _pad1(cycle_id.astype(jnp.int32), cp, BIG),
        zeros_h)

    gp = rup(C_ROWS, NSC * NTILE * GB)
    cyc_gath = _make_gather(gp)(
        cyc_sum, _pad1(cycle_id.astype(jnp.int32), gp, 0))

    h1, st1 = _mm_stats([cycle_rep, cyc_gath, e2c], cyc_W1, C_ROWS)
    ss1 = _scale_shift(st1, cyc_g1, cyc_b1, C_ROWS)
    h2, st2 = _bn_mm_stats(h1, ss1, cyc_W2, C_ROWS)
    ss2 = _scale_shift(st2, cyc_g2, cyc_b2, C_ROWS)
    cycle_out = _bn_apply(h2, ss2, C_ROWS)

    p2p = rup(P2, NTILE * IC)
    c2e = _make_scatter(p2p, E)(
        cycle_out,
        _pad1(c2e_src, p2p, 0),
        _pad1(c2e_dst, p2p, BIG),
        zeros_h)

    g1h, st3 = _mm_stats([edge_rep, c2e], edg_W1, E)
    ss3 = _scale_shift(st3, edg_g1, edg_b1, E)
    g2h, st4 = _bn_mm_stats(g1h, ss3, edg_W2, E)
    ss4 = _scale_shift(st4, edg_g2, edg_b2, E)
    edge_out = _bn_apply(g2h, ss4, E)

    return (edge_out, cycle_out)


# speedup vs baseline: 1.1324x; 1.0795x over previous
import functools

import jax
import jax.numpy as jnp
from jax import lax
from jax.experimental import pallas as pl
from jax.experimental.pallas import tpu as pltpu
from jax.experimental.pallas import tpu_sc as plsc

HID = 128
NSC = 2
NTILE = 16
CHUNK = 10112
NDUMP = 8
GB = 128
BIG = 1 << 30


def _pad1(x, n, val):
    return jnp.pad(x, (0, n - x.shape[0]), constant_values=val)


IC = 3840


def _make_scatter(n_pairs_pad, n_dest):
    assert n_pairs_pad % (NTILE * IC) == 0
    ppt = n_pairs_pad // NTILE
    nchunks = ppt // IC
    npass = -(-n_dest // (NSC * CHUNK))
    n_out_pad = npass * NSC * CHUNK
    zrows = CHUNK // NTILE
    mesh = plsc.VectorSubcoreMesh(core_axis_name="c", subcore_axis_name="s")

    @functools.partial(
        pl.kernel,
        out_type=jax.ShapeDtypeStruct((n_out_pad, HID), jnp.float32),
        mesh=mesh,
        scratch_types=[
            pltpu.VMEM((IC,), jnp.int32),
            pltpu.VMEM((IC,), jnp.int32),
            pltpu.VMEM((IC + 2 * GB,), jnp.int32),
            pltpu.VMEM((IC + 2 * GB,), jnp.int32),
            pltpu.VMEM((GB,), jnp.int32),
            pltpu.VMEM((GB, HID), jnp.float32),
            pltpu.VMEM((GB, HID), jnp.float32),
            pltpu.VMEM_SHARED((CHUNK + NDUMP, HID), jnp.float32),
            pltpu.SemaphoreType.DMA,
            pltpu.SemaphoreType.DMA,
        ],
        compiler_params=pltpu.CompilerParams(needs_layout_passes=False),
    )
    def k(table, srcs, dsts, zeros_h, out,
          srcv, dstv, msrc, mdst, idxb, rows0, rows1, shared, sem0, sem1):
        c = lax.axis_index("c")
        s = lax.axis_index("s")

        def one_pass(p, _):
            base = (p * NSC + c) * CHUNK
            for z in range(zrows // GB):
                pltpu.sync_copy(zeros_h,
                                shared.at[pl.ds(s * zrows + z * GB, GB)])
            if zrows % GB:
                rem = zrows % GB
                pltpu.sync_copy(
                    zeros_h.at[pl.ds(0, rem)],
                    shared.at[pl.ds(s * zrows + (zrows // GB) * GB, rem)])
            plsc.subcore_barrier()

            def chunk_body(kk, _):
                pltpu.sync_copy(srcs.at[pl.ds(s * ppt + kk * IC, IC)], srcv)
                pltpu.sync_copy(dsts.at[pl.ds(s * ppt + kk * IC, IC)], dstv)

                def scan(i, cnt):
                    dv = dstv[pl.ds(i * 16, 16)]
                    sv = srcv[pl.ds(i * 16, 16)]
                    lo = dv - base
                    m = (lo >= 0) & (lo < CHUNK)
                    mi = jnp.where(m, 1, 0)
                    csum = plsc.cumsum(mi)
                    pos = (cnt - 1) + csum
                    plsc.store_scatter(msrc, [pos], sv, mask=m)
                    plsc.store_scatter(mdst, [pos], lo, mask=m)
                    return cnt + jnp.squeeze(lax.slice(csum, (15,), (16,)))

                cnt = lax.fori_loop(0, IC // 16, scan, jnp.int32(0),
                                    unroll=4)
                for t in range(GB // 16):
                    lane = lax.iota(jnp.int32, 16)
                    msrc[pl.ds(cnt + t * 16, 16)] = (lane + s * 16) % 64
                    mdst[pl.ds(cnt + t * 16, 16)] = CHUNK + (lane + t) % NDUMP
                nb = (cnt + GB - 1) // GB

                @pl.when(nb > 0)
                def _():
                    pltpu.async_copy(
                        table.at[msrc.at[pl.ds(0, GB)]], rows0, sem0)

                def proc2(j2, _):
                    for par in range(2):
                        rbuf, rsem = (rows0, sem0) if par == 0 else (rows1, sem1)
                        obuf, osem = (rows1, sem1) if par == 0 else (rows0, sem0)
                        j = j2 * 2 + par

                        @pl.when(j < nb)
                        def _(j=j, rbuf=rbuf, rsem=rsem, obuf=obuf, osem=osem):
                            pltpu.make_async_copy(
                                table.at[msrc.at[pl.ds(j * GB, GB)]],
                                rbuf, rsem).wait()

                            @pl.when(j + 1 < nb)
                            def _():
                                pltpu.async_copy(
                                    table.at[msrc.at[pl.ds((j + 1) * GB, GB)]],
                                    obuf, osem)

                            for q in range(GB // 16):
                                idxb[pl.ds(q * 16, 16)] = (
                                    mdst[pl.ds(j * GB + q * 16, 16)])
                            pltpu.sync_copy(rbuf, shared.at[idxb], add=True)
                    return 0

                lax.fori_loop(0, (nb + 1) // 2, proc2, 0)
                return 0

            lax.fori_loop(0, nchunks, chunk_body, 0)
            plsc.subcore_barrier()
            pltpu.sync_copy(shared.at[pl.ds(s * zrows, zrows)],
                            out.at[pl.ds(base + s * zrows, zrows)])
            return 0

        lax.fori_loop(0, npass, one_pass, 0)

    return k


def _make_gather(n_rows_pad):
    assert n_rows_pad % (NSC * NTILE * GB) == 0
    per_w = n_rows_pad // (NSC * NTILE)
    nb = per_w // GB
    mesh = plsc.VectorSubcoreMesh(core_axis_name="c", subcore_axis_name="s")

    @functools.partial(
        pl.kernel,
        out_type=jax.ShapeDtypeStruct((n_rows_pad, HID), jnp.float32),
        mesh=mesh,
        scratch_types=[
            pltpu.VMEM((per_w,), jnp.int32),
            pltpu.VMEM((GB, HID), jnp.float32),
            pltpu.VMEM((GB, HID), jnp.float32),
            pltpu.SemaphoreType.DMA,
            pltpu.SemaphoreType.DMA,
        ],
        compiler_params=pltpu.CompilerParams(needs_layout_passes=False),
    )
    def k(table, idx, out, idxv, rows0, rows1, sem0, sem1):
        c = lax.axis_index("c")
        s = lax.axis_index("s")
        base = (s * NSC + c) * per_w
        pltpu.sync_copy(idx.at[pl.ds(base, per_w)], idxv)
        pltpu.async_copy(table.at[idxv.at[pl.ds(0, GB)]], rows0, sem0)

        def body(j2, _):
            for par in range(2):
                rbuf, rsem = (rows0, sem0) if par == 0 else (rows1, sem1)
                obuf, osem = (rows1, sem1) if par == 0 else (rows0, sem0)
                j = j2 * 2 + par

                @pl.when(j < nb)
                def _(j=j, rbuf=rbuf, rsem=rsem, obuf=obuf, osem=osem):
                    pltpu.make_async_copy(
                        table.at[idxv.at[pl.ds(j * GB, GB)]], rbuf, rsem
                    ).wait()

                    @pl.when(j + 1 < nb)
                    def _():
                        pltpu.async_copy(
                            table.at[idxv.at[pl.ds((j + 1) * GB, GB)]],
                            obuf, osem)

                    pltpu.sync_copy(rbuf, out.at[pl.ds(base + j * GB, GB)])
            return 0

        lax.fori_loop(0, (nb + 1) // 2, body, 0)

    return k


_BLK = 2048


def _mm_plain(x, w, n_valid):
    cout = w.shape[1]
    nb = pl.cdiv(n_valid, _BLK)

    def body(x_ref, w_ref, out_ref):
        out_ref[...] = jnp.dot(x_ref[...], w_ref[...],
                               preferred_element_type=jnp.float32)

    return pl.pallas_call(
        body,
        grid=(nb,),
        in_specs=[pl.BlockSpec((_BLK, x.shape[1]), lambda i: (i, 0)),
                  pl.BlockSpec(w.shape, lambda i: (0, 0))],
        out_specs=pl.BlockSpec((_BLK, cout), lambda i: (i, 0)),
        out_shape=jax.ShapeDtypeStruct((nb * _BLK, cout), jnp.float32),
    )(x, w)


def _mm_stats(parts, w, n_valid, pre=None):
    npart = len(parts)
    cout = w.shape[1]
    nb = pl.cdiv(n_valid, _BLK)
    has_pre = pre is not None
    ins = list(parts) + ([pre] if has_pre else [])

    def body(*refs):
        part_refs = refs[:npart]
        pre_ref = refs[npart] if has_pre else None
        w_ref, out_ref, st_ref = refs[npart + has_pre:]
        i = pl.program_id(0)
        rowid = lax.broadcasted_iota(jnp.int32, (_BLK, 1), 0) + i * _BLK
        valid = rowid < n_valid
        h = pre_ref[...] if has_pre else jnp.zeros((_BLK, cout), jnp.float32)
        for kk in range(npart):
            h = h + jnp.dot(part_refs[kk][...],
                            w_ref[kk * HID:(kk + 1) * HID, :],
                            preferred_element_type=jnp.float32)
        h = jnp.where(valid, h, 0.0)
        out_ref[...] = h
        su = jnp.sum(h, axis=0, keepdims=True)
        sq = jnp.sum(h * h, axis=0, keepdims=True)
        upd = jnp.concatenate([su, sq, jnp.zeros((6, cout), jnp.float32)], 0)

        @pl.when(i == 0)
        def _():
            st_ref[...] = upd

        @pl.when(i > 0)
        def _():
            st_ref[...] += upd

    out, st = pl.pallas_call(
        body,
        grid=(nb,),
        in_specs=[pl.BlockSpec((_BLK, HID), lambda i: (i, 0))] * npart
        + ([pl.BlockSpec((_BLK, cout), lambda i: (i, 0))] if has_pre else [])
        + [pl.BlockSpec(w.shape, lambda i: (0, 0))],
        out_specs=[pl.BlockSpec((_BLK, cout), lambda i: (i, 0)),
                   pl.BlockSpec((8, cout), lambda i: (0, 0))],
        out_shape=[jax.ShapeDtypeStruct((nb * _BLK, cout), jnp.float32),
                   jax.ShapeDtypeStruct((8, cout), jnp.float32)],
    )(*ins, w)
    return out, st


def _bn_mm_stats(x, ss, w, n_valid):
    cout = w.shape[1]
    nb = x.shape[0] // _BLK

    def body(x_ref, ss_ref, w_ref, out_ref, st_ref):
        i = pl.program_id(0)
        rowid = lax.broadcasted_iota(jnp.int32, (_BLK, 1), 0) + i * _BLK
        valid = rowid < n_valid
        a = jnp.maximum(x_ref[...] * ss_ref[0:1, :] + ss_ref[1:2, :], 0.0)
        a = jnp.where(valid, a, 0.0)
        h = jnp.dot(a, w_ref[...], preferred_element_type=jnp.float32)
        out_ref[...] = h
        su = jnp.sum(h, axis=0, keepdims=True)
        sq = jnp.sum(h * h, axis=0, keepdims=True)
        upd = jnp.concatenate([su, sq, jnp.zeros((6, cout), jnp.float32)], 0)

        @pl.when(i == 0)
        def _():
            st_ref[...] = upd

        @pl.when(i > 0)
        def _():
            st_ref[...] += upd

    return pl.pallas_call(
        body,
        grid=(nb,),
        in_specs=[pl.BlockSpec((_BLK, x.shape[1]), lambda i: (i, 0)),
                  pl.BlockSpec(ss.shape, lambda i: (0, 0)),
                  pl.BlockSpec(w.shape, lambda i: (0, 0))],
        out_specs=[pl.BlockSpec((_BLK, cout), lambda i: (i, 0)),
                   pl.BlockSpec((8, cout), lambda i: (0, 0))],
        out_shape=[jax.ShapeDtypeStruct((nb * _BLK, cout), jnp.float32),
                   jax.ShapeDtypeStruct((8, cout), jnp.float32)],
    )(x, ss, w)


def _bn_apply(x, ss, n_valid):
    c = x.shape[1]
    blk = 1024
    nb = pl.cdiv(n_valid, blk)

    def body(x_ref, ss_ref, out_ref):
        out_ref[...] = jnp.maximum(
            x_ref[...] * ss_ref[0:1, :] + ss_ref[1:2, :], 0.0)

    return pl.pallas_call(
        body,
        grid=(nb,),
        in_specs=[pl.BlockSpec((blk, c), lambda i: (i, 0)),
                  pl.BlockSpec(ss.shape, lambda i: (0, 0))],
        out_specs=pl.BlockSpec((blk, c), lambda i: (i, 0)),
        out_shape=jax.ShapeDtypeStruct((n_valid, c), jnp.float32),
    )(x, ss)


def _scale_shift(st, g, b, n):
    mu = st[0] / n
    var = st[1] / n - mu * mu
    scale = g * lax.rsqrt(var + 1e-5)
    shift = b - mu * scale
    return jnp.concatenate(
        [scale[None], shift[None], jnp.zeros((6, scale.shape[0]), jnp.float32)], 0)


def kernel(edge_rep, cycle_rep, e2c_src, e2c_dst, cycle_id, c2e_src, c2e_dst,
           cyc_W1, cyc_g1, cyc_b1, cyc_W2, cyc_g2, cyc_b2,
           edg_W1, edg_g1, edg_b1, edg_W2, edg_g2, edg_b2):
    E, _ = edge_rep.shape
    C_ROWS, _ = cycle_rep.shape
    P1 = e2c_src.shape[0]
    P2 = c2e_src.shape[0]
    N_CYC = 30000

    zeros_h = jnp.zeros((GB, HID), jnp.float32)

    def rup(n, m):
        return (n + m - 1) // m * m

    p1p = rup(P1, NTILE * IC)
    e2c = _make_scatter(p1p, C_ROWS)(
        edge_rep,
        _pad1(e2c_src, p1p, 0),
        _pad1(e2c_dst, p1p, BIG),
        zeros_h)

    cp = rup(C_ROWS, NTILE * IC)
    cyc_sum = _make_scatter(cp, N_CYC)(
        e2c,
        _pad1(jnp.arange(C_ROWS, dtype=jnp.int32), cp, 0),
        _pad1(cycle_id.astype(jnp.int32), cp, BIG),
        zeros_h)

    gp = rup(C_ROWS, NSC * NTILE * GB)
    cyc_gath = _make_gather(gp)(
        cyc_sum, _pad1(cycle_id.astype(jnp.int32), gp, 0))

    h1, st1 = _mm_stats([cycle_rep, cyc_gath, e2c], cyc_W1, C_ROWS)
    ss1 = _scale_shift(st1, cyc_g1, cyc_b1, C_ROWS)
    h2, st2 = _bn_mm_stats(h1, ss1, cyc_W2, C_ROWS)
    ss2 = _scale_shift(st2, cyc_g2, cyc_b2, C_ROWS)
    cycle_out = _bn_apply(h2, ss2, C_ROWS)

    p2p = rup(P2, NTILE * IC)
    c2e = _make_scatter(p2p, E)(
        cycle_out,
        _pad1(c2e_src, p2p, 0),
        _pad1(c2e_dst, p2p, BIG),
        zeros_h)

    g1h, st3 = _mm_stats([edge_rep, c2e], edg_W1, E)
    ss3 = _scale_shift(st3, edg_g1, edg_b1, E)
    g2h, st4 = _bn_mm_stats(g1h, ss3, edg_W2, E)
    ss4 = _scale_shift(st4, edg_g2, edg_b2, E)
    edge_out = _bn_apply(g2h, ss4, E)

    return (edge_out, cycle_out)


# speedup vs baseline: 1.1821x; 1.0439x over previous
import functools

import jax
import jax.numpy as jnp
from jax import lax
from jax.experimental import pallas as pl
from jax.experimental.pallas import tpu as pltpu
from jax.experimental.pallas import tpu_sc as plsc

HID = 128
NSC = 2
NTILE = 16
CHUNK = 10112
NDUMP = 8
GB = 128
BIG = 1 << 30


def _pad1(x, n, val):
    return jnp.pad(x, (0, n - x.shape[0]), constant_values=val)


IC = 3840


def _make_scatter(n_pairs_pad, n_dest):
    assert n_pairs_pad % (NTILE * IC) == 0
    ppt = n_pairs_pad // NTILE
    nchunks = ppt // IC
    npass = -(-n_dest // (NSC * CHUNK))
    n_out_pad = npass * NSC * CHUNK
    zrows = CHUNK // NTILE
    mesh = plsc.VectorSubcoreMesh(core_axis_name="c", subcore_axis_name="s")

    @functools.partial(
        pl.kernel,
        out_type=jax.ShapeDtypeStruct((n_out_pad, HID), jnp.float32),
        mesh=mesh,
        scratch_types=[
            pltpu.VMEM((IC,), jnp.int32),
            pltpu.VMEM((IC,), jnp.int32),
            pltpu.VMEM((IC + 2 * GB,), jnp.int32),
            pltpu.VMEM((IC + 2 * GB,), jnp.int32),
            pltpu.VMEM((GB,), jnp.int32),
            pltpu.VMEM((GB, HID), jnp.float32),
            pltpu.VMEM((GB, HID), jnp.float32),
            pltpu.VMEM_SHARED((CHUNK + NDUMP, HID), jnp.float32),
            pltpu.SemaphoreType.DMA,
            pltpu.SemaphoreType.DMA,
        ],
        compiler_params=pltpu.CompilerParams(needs_layout_passes=False),
    )
    def k(table, srcs, dsts, zeros_h, out,
          srcv, dstv, msrc, mdst, idxb, rows0, rows1, shared, sem0, sem1):
        c = lax.axis_index("c")
        s = lax.axis_index("s")

        def one_pass(p, _):
            base = (p * NSC + c) * CHUNK
            for z in range(zrows // GB):
                pltpu.sync_copy(zeros_h,
                                shared.at[pl.ds(s * zrows + z * GB, GB)])
            if zrows % GB:
                rem = zrows % GB
                pltpu.sync_copy(
                    zeros_h.at[pl.ds(0, rem)],
                    shared.at[pl.ds(s * zrows + (zrows // GB) * GB, rem)])
            plsc.subcore_barrier()

            def chunk_body(kk, _):
                pltpu.sync_copy(srcs.at[pl.ds(s * ppt + kk * IC, IC)], srcv)
                pltpu.sync_copy(dsts.at[pl.ds(s * ppt + kk * IC, IC)], dstv)

                def scan(i, cnt):
                    dv = dstv[pl.ds(i * 16, 16)]
                    sv = srcv[pl.ds(i * 16, 16)]
                    lo = dv - base
                    m = (lo >= 0) & (lo < CHUNK)
                    mi = jnp.where(m, 1, 0)
                    csum = plsc.cumsum(mi)
                    pos = (cnt - 1) + csum
                    plsc.store_scatter(msrc, [pos], sv, mask=m)
                    plsc.store_scatter(mdst, [pos], lo, mask=m)
                    return cnt + jnp.squeeze(lax.slice(csum, (15,), (16,)))

                cnt = lax.fori_loop(0, IC // 16, scan, jnp.int32(0),
                                    unroll=4)
                for t in range(GB // 16):
                    lane = lax.iota(jnp.int32, 16)
                    msrc[pl.ds(cnt + t * 16, 16)] = (lane + s * 16) % 64
                    mdst[pl.ds(cnt + t * 16, 16)] = CHUNK + (lane + t) % NDUMP
                nb = (cnt + GB - 1) // GB

                @pl.when(nb > 0)
                def _():
                    pltpu.async_copy(
                        table.at[msrc.at[pl.ds(0, GB)]], rows0, sem0)

                def proc2(j2, _):
                    for par in range(2):
                        rbuf, rsem = (rows0, sem0) if par == 0 else (rows1, sem1)
                        obuf, osem = (rows1, sem1) if par == 0 else (rows0, sem0)
                        j = j2 * 2 + par

                        @pl.when(j < nb)
                        def _(j=j, rbuf=rbuf, rsem=rsem, obuf=obuf, osem=osem):
                            pltpu.make_async_copy(
                                table.at[msrc.at[pl.ds(j * GB, GB)]],
                                rbuf, rsem).wait()

                            @pl.when(j + 1 < nb)
                            def _():
                                pltpu.async_copy(
                                    table.at[msrc.at[pl.ds((j + 1) * GB, GB)]],
                                    obuf, osem)

                            for q in range(GB // 16):
                                idxb[pl.ds(q * 16, 16)] = (
                                    mdst[pl.ds(j * GB + q * 16, 16)])
                            pltpu.sync_copy(rbuf, shared.at[idxb], add=True)
                    return 0

                lax.fori_loop(0, (nb + 1) // 2, proc2, 0)
                return 0

            lax.fori_loop(0, nchunks, chunk_body, 0)
            plsc.subcore_barrier()
            pltpu.sync_copy(shared.at[pl.ds(s * zrows, zrows)],
                            out.at[pl.ds(base + s * zrows, zrows)])
            return 0

        lax.fori_loop(0, npass, one_pass, 0)

    return k


def _make_gather(n_rows_pad):
    assert n_rows_pad % (NSC * NTILE * GB) == 0
    per_w = n_rows_pad // (NSC * NTILE)
    nb = per_w // GB
    mesh = plsc.VectorSubcoreMesh(core_axis_name="c", subcore_axis_name="s")

    @functools.partial(
        pl.kernel,
        out_type=jax.ShapeDtypeStruct((n_rows_pad, HID), jnp.float32),
        mesh=mesh,
        scratch_types=[
            pltpu.VMEM((per_w,), jnp.int32),
            pltpu.VMEM((GB, HID), jnp.float32),
            pltpu.VMEM((GB, HID), jnp.float32),
            pltpu.SemaphoreType.DMA,
            pltpu.SemaphoreType.DMA,
        ],
        compiler_params=pltpu.CompilerParams(needs_layout_passes=False),
    )
    def k(table, idx, out, idxv, rows0, rows1, sem0, sem1):
        c = lax.axis_index("c")
        s = lax.axis_index("s")
        base = (s * NSC + c) * per_w
        pltpu.sync_copy(idx.at[pl.ds(base, per_w)], idxv)
        pltpu.async_copy(table.at[idxv.at[pl.ds(0, GB)]], rows0, sem0)

        def body(j2, _):
            for par in range(2):
                rbuf, rsem = (rows0, sem0) if par == 0 else (rows1, sem1)
                obuf, osem = (rows1, sem1) if par == 0 else (rows0, sem0)
                j = j2 * 2 + par

                @pl.when(j < nb)
                def _(j=j, rbuf=rbuf, rsem=rsem, obuf=obuf, osem=osem):
                    pltpu.make_async_copy(
                        table.at[idxv.at[pl.ds(j * GB, GB)]], rbuf, rsem
                    ).wait()

                    @pl.when(j + 1 < nb)
                    def _():
                        pltpu.async_copy(
                            table.at[idxv.at[pl.ds((j + 1) * GB, GB)]],
                            obuf, osem)

                    pltpu.sync_copy(rbuf, out.at[pl.ds(base + j * GB, GB)])
            return 0

        lax.fori_loop(0, (nb + 1) // 2, body, 0)

    return k


_BLK = 4096


def _mm_plain(x, w, n_valid):
    cout = w.shape[1]
    nb = pl.cdiv(n_valid, _BLK)

    def body(x_ref, w_ref, out_ref):
        out_ref[...] = jnp.dot(x_ref[...], w_ref[...],
                               preferred_element_type=jnp.float32)

    return pl.pallas_call(
        body,
        grid=(nb,),
        in_specs=[pl.BlockSpec((_BLK, x.shape[1]), lambda i: (i, 0)),
                  pl.BlockSpec(w.shape, lambda i: (0, 0))],
        out_specs=pl.BlockSpec((_BLK, cout), lambda i: (i, 0)),
        out_shape=jax.ShapeDtypeStruct((nb * _BLK, cout), jnp.float32),
    )(x, w)


def _mm_stats(parts, w, n_valid, pre=None):
    npart = len(parts)
    cout = w.shape[1]
    nb = pl.cdiv(n_valid, _BLK)
    has_pre = pre is not None
    ins = list(parts) + ([pre] if has_pre else [])

    def body(*refs):
        part_refs = refs[:npart]
        pre_ref = refs[npart] if has_pre else None
        w_ref, out_ref, st_ref = refs[npart + has_pre:]
        i = pl.program_id(0)
        rowid = lax.broadcasted_iota(jnp.int32, (_BLK, 1), 0) + i * _BLK
        valid = rowid < n_valid
        h = pre_ref[...] if has_pre else jnp.zeros((_BLK, cout), jnp.float32)
        for kk in range(npart):
            h = h + jnp.dot(part_refs[kk][...],
                            w_ref[kk * HID:(kk + 1) * HID, :],
                            preferred_element_type=jnp.float32)
        h = jnp.where(valid, h, 0.0)
        out_ref[...] = h
        su = jnp.sum(h, axis=0, keepdims=True)
        sq = jnp.sum(h * h, axis=0, keepdims=True)
        upd = jnp.concatenate([su, sq, jnp.zeros((6, cout), jnp.float32)], 0)

        @pl.when(i == 0)
        def _():
            st_ref[...] = upd

        @pl.when(i > 0)
        def _():
            st_ref[...] += upd

    out, st = pl.pallas_call(
        body,
        grid=(nb,),
        in_specs=[pl.BlockSpec((_BLK, HID), lambda i: (i, 0))] * npart
        + ([pl.BlockSpec((_BLK, cout), lambda i: (i, 0))] if has_pre else [])
        + [pl.BlockSpec(w.shape, lambda i: (0, 0))],
        out_specs=[pl.BlockSpec((_BLK, cout), lambda i: (i, 0)),
                   pl.BlockSpec((8, cout), lambda i: (0, 0))],
        out_shape=[jax.ShapeDtypeStruct((nb * _BLK, cout), jnp.float32),
                   jax.ShapeDtypeStruct((8, cout), jnp.float32)],
    )(*ins, w)
    return out, st


def _bn_mm_stats(x, ss, w, n_valid):
    cout = w.shape[1]
    nb = x.shape[0] // _BLK

    def body(x_ref, ss_ref, w_ref, out_ref, st_ref):
        i = pl.program_id(0)
        rowid = lax.broadcasted_iota(jnp.int32, (_BLK, 1), 0) + i * _BLK
        valid = rowid < n_valid
        a = jnp.maximum(x_ref[...] * ss_ref[0:1, :] + ss_ref[1:2, :], 0.0)
        a = jnp.where(valid, a, 0.0)
        h = jnp.dot(a, w_ref[...], preferred_element_type=jnp.float32)
        out_ref[...] = h
        su = jnp.sum(h, axis=0, keepdims=True)
        sq = jnp.sum(h * h, axis=0, keepdims=True)
        upd = jnp.concatenate([su, sq, jnp.zeros((6, cout), jnp.float32)], 0)

        @pl.when(i == 0)
        def _():
            st_ref[...] = upd

        @pl.when(i > 0)
        def _():
            st_ref[...] += upd

    return pl.pallas_call(
        body,
        grid=(nb,),
        in_specs=[pl.BlockSpec((_BLK, x.shape[1]), lambda i: (i, 0)),
                  pl.BlockSpec(ss.shape, lambda i: (0, 0)),
                  pl.BlockSpec(w.shape, lambda i: (0, 0))],
        out_specs=[pl.BlockSpec((_BLK, cout), lambda i: (i, 0)),
                   pl.BlockSpec((8, cout), lambda i: (0, 0))],
        out_shape=[jax.ShapeDtypeStruct((nb * _BLK, cout), jnp.float32),
                   jax.ShapeDtypeStruct((8, cout), jnp.float32)],
    )(x, ss, w)


def _bn_apply(x, ss, n_valid):
    c = x.shape[1]
    blk = 1024
    nb = pl.cdiv(n_valid, blk)

    def body(x_ref, ss_ref, out_ref):
        out_ref[...] = jnp.maximum(
            x_ref[...] * ss_ref[0:1, :] + ss_ref[1:2, :], 0.0)

    return pl.pallas_call(
        body,
        grid=(nb,),
        in_specs=[pl.BlockSpec((blk, c), lambda i: (i, 0)),
                  pl.BlockSpec(ss.shape, lambda i: (0, 0))],
        out_specs=pl.BlockSpec((blk, c), lambda i: (i, 0)),
        out_shape=jax.ShapeDtypeStruct((n_valid, c), jnp.float32),
    )(x, ss)


def _scale_shift(st, g, b, n):
    mu = st[0] / n
    var = st[1] / n - mu * mu
    scale = g * lax.rsqrt(var + 1e-5)
    shift = b - mu * scale
    return jnp.concatenate(
        [scale[None], shift[None], jnp.zeros((6, scale.shape[0]), jnp.float32)], 0)


def kernel(edge_rep, cycle_rep, e2c_src, e2c_dst, cycle_id, c2e_src, c2e_dst,
           cyc_W1, cyc_g1, cyc_b1, cyc_W2, cyc_g2, cyc_b2,
           edg_W1, edg_g1, edg_b1, edg_W2, edg_g2, edg_b2):
    E, _ = edge_rep.shape
    C_ROWS, _ = cycle_rep.shape
    P1 = e2c_src.shape[0]
    P2 = c2e_src.shape[0]
    N_CYC = 30000

    zeros_h = jnp.zeros((GB, HID), jnp.float32)

    def rup(n, m):
        return (n + m - 1) // m * m

    p1p = rup(P1, NTILE * IC)
    e2c = _make_scatter(p1p, C_ROWS)(
        edge_rep,
        _pad1(e2c_src, p1p, 0),
        _pad1(e2c_dst, p1p, BIG),
        zeros_h)

    cp = rup(C_ROWS, NTILE * IC)
    cyc_sum = _make_scatter(cp, N_CYC)(
        e2c,
        _pad1(jnp.arange(C_ROWS, dtype=jnp.int32), cp, 0),
        _pad1(cycle_id.astype(jnp.int32), cp, BIG),
        zeros_h)

    gp = rup(C_ROWS, NSC * NTILE * GB)
    cyc_gath = _make_gather(gp)(
        cyc_sum, _pad1(cycle_id.astype(jnp.int32), gp, 0))

    h1, st1 = _mm_stats([cycle_rep, cyc_gath, e2c], cyc_W1, C_ROWS)
    ss1 = _scale_shift(st1, cyc_g1, cyc_b1, C_ROWS)
    h2, st2 = _bn_mm_stats(h1, ss1, cyc_W2, C_ROWS)
    ss2 = _scale_shift(st2, cyc_g2, cyc_b2, C_ROWS)
    cycle_out = _bn_apply(h2, ss2, C_ROWS)

    p2p = rup(P2, NTILE * IC)
    c2e = _make_scatter(p2p, E)(
        cycle_out,
        _pad1(c2e_src, p2p, 0),
        _pad1(c2e_dst, p2p, BIG),
        zeros_h)

    g1h, st3 = _mm_stats([edge_rep, c2e], edg_W1, E)
    ss3 = _scale_shift(st3, edg_g1, edg_b1, E)
    g2h, st4 = _bn_mm_stats(g1h, ss3, edg_W2, E)
    ss4 = _scale_shift(st4, edg_g2, edg_b2, E)
    edge_out = _bn_apply(g2h, ss4, E)

    return (edge_out, cycle_out)


# speedup vs baseline: 1.1872x; 1.0043x over previous
import functools

import jax
import jax.numpy as jnp
from jax import lax
from jax.experimental import pallas as pl
from jax.experimental.pallas import tpu as pltpu
from jax.experimental.pallas import tpu_sc as plsc

HID = 128
NSC = 2
NTILE = 16
CHUNK = 10112
NDUMP = 8
GB = 128
BIG = 1 << 30


def _pad1(x, n, val):
    return jnp.pad(x, (0, n - x.shape[0]), constant_values=val)


IC = 3840


def _make_scatter(n_pairs_pad, n_dest):
    assert n_pairs_pad % (NTILE * IC) == 0
    ppt = n_pairs_pad // NTILE
    nchunks = ppt // IC
    npass = -(-n_dest // (NSC * CHUNK))
    n_out_pad = npass * NSC * CHUNK
    zrows = CHUNK // NTILE
    mesh = plsc.VectorSubcoreMesh(core_axis_name="c", subcore_axis_name="s")

    @functools.partial(
        pl.kernel,
        out_type=jax.ShapeDtypeStruct((n_out_pad, HID), jnp.float32),
        mesh=mesh,
        scratch_types=[
            pltpu.VMEM((IC,), jnp.int32),
            pltpu.VMEM((IC,), jnp.int32),
            pltpu.VMEM((IC + 2 * GB,), jnp.int32),
            pltpu.VMEM((IC + 2 * GB,), jnp.int32),
            pltpu.VMEM((GB,), jnp.int32),
            pltpu.VMEM((GB, HID), jnp.float32),
            pltpu.VMEM((GB, HID), jnp.float32),
            pltpu.VMEM_SHARED((CHUNK + NDUMP, HID), jnp.float32),
            pltpu.SemaphoreType.DMA,
            pltpu.SemaphoreType.DMA,
        ],
        compiler_params=pltpu.CompilerParams(needs_layout_passes=False),
    )
    def k(table, srcs, dsts, zeros_h, out,
          srcv, dstv, msrc, mdst, idxb, rows0, rows1, shared, sem0, sem1):
        c = lax.axis_index("c")
        s = lax.axis_index("s")

        def one_pass(p, _):
            base = (p * NSC + c) * CHUNK
            for z in range(zrows // GB):
                pltpu.sync_copy(zeros_h,
                                shared.at[pl.ds(s * zrows + z * GB, GB)])
            if zrows % GB:
                rem = zrows % GB
                pltpu.sync_copy(
                    zeros_h.at[pl.ds(0, rem)],
                    shared.at[pl.ds(s * zrows + (zrows // GB) * GB, rem)])
            plsc.subcore_barrier()

            def chunk_body(kk, _):
                pltpu.sync_copy(srcs.at[pl.ds(s * ppt + kk * IC, IC)], srcv)
                pltpu.sync_copy(dsts.at[pl.ds(s * ppt + kk * IC, IC)], dstv)

                def scan(i, cnt):
                    dv = dstv[pl.ds(i * 16, 16)]
                    sv = srcv[pl.ds(i * 16, 16)]
                    lo = dv - base
                    m = (lo >= 0) & (lo < CHUNK)
                    mi = jnp.where(m, 1, 0)
                    csum = plsc.cumsum(mi)
                    pos = (cnt - 1) + csum
                    plsc.store_scatter(msrc, [pos], sv, mask=m)
                    plsc.store_scatter(mdst, [pos], lo, mask=m)
                    return cnt + jnp.squeeze(lax.slice(csum, (15,), (16,)))

                cnt = lax.fori_loop(0, IC // 16, scan, jnp.int32(0),
                                    unroll=4)
                for t in range(GB // 16):
                    lane = lax.iota(jnp.int32, 16)
                    msrc[pl.ds(cnt + t * 16, 16)] = (lane + s * 16) % 64
                    mdst[pl.ds(cnt + t * 16, 16)] = CHUNK + (lane + t) % NDUMP
                nb = (cnt + GB - 1) // GB

                @pl.when(nb > 0)
                def _():
                    pltpu.async_copy(
                        table.at[msrc.at[pl.ds(0, GB)]], rows0, sem0)

                def proc2(j2, _):
                    for par in range(2):
                        rbuf, rsem = (rows0, sem0) if par == 0 else (rows1, sem1)
                        obuf, osem = (rows1, sem1) if par == 0 else (rows0, sem0)
                        j = j2 * 2 + par

                        @pl.when(j < nb)
                        def _(j=j, rbuf=rbuf, rsem=rsem, obuf=obuf, osem=osem):
                            pltpu.make_async_copy(
                                table.at[msrc.at[pl.ds(j * GB, GB)]],
                                rbuf, rsem).wait()

                            @pl.when(j + 1 < nb)
                            def _():
                                pltpu.async_copy(
                                    table.at[msrc.at[pl.ds((j + 1) * GB, GB)]],
                                    obuf, osem)

                            for q in range(GB // 16):
                                idxb[pl.ds(q * 16, 16)] = (
                                    mdst[pl.ds(j * GB + q * 16, 16)])
                            pltpu.sync_copy(rbuf, shared.at[idxb], add=True)
                    return 0

                lax.fori_loop(0, (nb + 1) // 2, proc2, 0)
                return 0

            lax.fori_loop(0, nchunks, chunk_body, 0)
            plsc.subcore_barrier()
            pltpu.sync_copy(shared.at[pl.ds(s * zrows, zrows)],
                            out.at[pl.ds(base + s * zrows, zrows)])
            return 0

        lax.fori_loop(0, npass, one_pass, 0)

    return k


def _make_gather(n_rows_pad):
    assert n_rows_pad % (NSC * NTILE * GB) == 0
    per_w = n_rows_pad // (NSC * NTILE)
    nb = per_w // GB
    mesh = plsc.VectorSubcoreMesh(core_axis_name="c", subcore_axis_name="s")

    @functools.partial(
        pl.kernel,
        out_type=jax.ShapeDtypeStruct((n_rows_pad, HID), jnp.float32),
        mesh=mesh,
        scratch_types=[
            pltpu.VMEM((per_w,), jnp.int32),
            pltpu.VMEM((GB, HID), jnp.float32),
            pltpu.VMEM((GB, HID), jnp.float32),
            pltpu.SemaphoreType.DMA,
            pltpu.SemaphoreType.DMA,
        ],
        compiler_params=pltpu.CompilerParams(needs_layout_passes=False),
    )
    def k(table, idx, out, idxv, rows0, rows1, sem0, sem1):
        c = lax.axis_index("c")
        s = lax.axis_index("s")
        base = (s * NSC + c) * per_w
        pltpu.sync_copy(idx.at[pl.ds(base, per_w)], idxv)
        pltpu.async_copy(table.at[idxv.at[pl.ds(0, GB)]], rows0, sem0)

        def body(j2, _):
            for par in range(2):
                rbuf, rsem = (rows0, sem0) if par == 0 else (rows1, sem1)
                obuf, osem = (rows1, sem1) if par == 0 else (rows0, sem0)
                j = j2 * 2 + par

                @pl.when(j < nb)
                def _(j=j, rbuf=rbuf, rsem=rsem, obuf=obuf, osem=osem):
                    pltpu.make_async_copy(
                        table.at[idxv.at[pl.ds(j * GB, GB)]], rbuf, rsem
                    ).wait()

                    @pl.when(j + 1 < nb)
                    def _():
                        pltpu.async_copy(
                            table.at[idxv.at[pl.ds((j + 1) * GB, GB)]],
                            obuf, osem)

                    pltpu.sync_copy(rbuf, out.at[pl.ds(base + j * GB, GB)])
            return 0

        lax.fori_loop(0, (nb + 1) // 2, body, 0)

    return k


_BLK = 8192


def _mm_plain(x, w, n_valid):
    cout = w.shape[1]
    nb = pl.cdiv(n_valid, _BLK)

    def body(x_ref, w_ref, out_ref):
        out_ref[...] = jnp.dot(x_ref[...], w_ref[...],
                               preferred_element_type=jnp.float32)

    return pl.pallas_call(
        body,
        grid=(nb,),
        in_specs=[pl.BlockSpec((_BLK, x.shape[1]), lambda i: (i, 0)),
                  pl.BlockSpec(w.shape, lambda i: (0, 0))],
        out_specs=pl.BlockSpec((_BLK, cout), lambda i: (i, 0)),
        out_shape=jax.ShapeDtypeStruct((nb * _BLK, cout), jnp.float32),
    )(x, w)


def _mm_stats(parts, w, n_valid, pre=None):
    npart = len(parts)
    cout = w.shape[1]
    nb = pl.cdiv(n_valid, _BLK)
    has_pre = pre is not None
    ins = list(parts) + ([pre] if has_pre else [])

    def body(*refs):
        part_refs = refs[:npart]
        pre_ref = refs[npart] if has_pre else None
        w_ref, out_ref, st_ref = refs[npart + has_pre:]
        i = pl.program_id(0)
        rowid = lax.broadcasted_iota(jnp.int32, (_BLK, 1), 0) + i * _BLK
        valid = rowid < n_valid
        h = pre_ref[...] if has_pre else jnp.zeros((_BLK, cout), jnp.float32)
        for kk in range(npart):
            h = h + jnp.dot(part_refs[kk][...],
                            w_ref[kk * HID:(kk + 1) * HID, :],
                            preferred_element_type=jnp.float32)
        h = jnp.where(valid, h, 0.0)
        out_ref[...] = h
        su = jnp.sum(h, axis=0, keepdims=True)
        sq = jnp.sum(h * h, axis=0, keepdims=True)
        upd = jnp.concatenate([su, sq, jnp.zeros((6, cout), jnp.float32)], 0)

        @pl.when(i == 0)
        def _():
            st_ref[...] = upd

        @pl.when(i > 0)
        def _():
            st_ref[...] += upd

    out, st = pl.pallas_call(
        body,
        grid=(nb,),
        in_specs=[pl.BlockSpec((_BLK, HID), lambda i: (i, 0))] * npart
        + ([pl.BlockSpec((_BLK, cout), lambda i: (i, 0))] if has_pre else [])
        + [pl.BlockSpec(w.shape, lambda i: (0, 0))],
        out_specs=[pl.BlockSpec((_BLK, cout), lambda i: (i, 0)),
                   pl.BlockSpec((8, cout), lambda i: (0, 0))],
        out_shape=[jax.ShapeDtypeStruct((nb * _BLK, cout), jnp.float32),
                   jax.ShapeDtypeStruct((8, cout), jnp.float32)],
    )(*ins, w)
    return out, st


def _bn_mm_stats(x, ss, w, n_valid):
    cout = w.shape[1]
    nb = x.shape[0] // _BLK

    def body(x_ref, ss_ref, w_ref, out_ref, st_ref):
        i = pl.program_id(0)
        rowid = lax.broadcasted_iota(jnp.int32, (_BLK, 1), 0) + i * _BLK
        valid = rowid < n_valid
        a = jnp.maximum(x_ref[...] * ss_ref[0:1, :] + ss_ref[1:2, :], 0.0)
        a = jnp.where(valid, a, 0.0)
        h = jnp.dot(a, w_ref[...], preferred_element_type=jnp.float32)
        out_ref[...] = h
        su = jnp.sum(h, axis=0, keepdims=True)
        sq = jnp.sum(h * h, axis=0, keepdims=True)
        upd = jnp.concatenate([su, sq, jnp.zeros((6, cout), jnp.float32)], 0)

        @pl.when(i == 0)
        def _():
            st_ref[...] = upd

        @pl.when(i > 0)
        def _():
            st_ref[...] += upd

    return pl.pallas_call(
        body,
        grid=(nb,),
        in_specs=[pl.BlockSpec((_BLK, x.shape[1]), lambda i: (i, 0)),
                  pl.BlockSpec(ss.shape, lambda i: (0, 0)),
                  pl.BlockSpec(w.shape, lambda i: (0, 0))],
        out_specs=[pl.BlockSpec((_BLK, cout), lambda i: (i, 0)),
                   pl.BlockSpec((8, cout), lambda i: (0, 0))],
        out_shape=[jax.ShapeDtypeStruct((nb * _BLK, cout), jnp.float32),
                   jax.ShapeDtypeStruct((8, cout), jnp.float32)],
    )(x, ss, w)


def _bn_apply(x, ss, n_valid):
    c = x.shape[1]
    blk = 1024
    nb = pl.cdiv(n_valid, blk)

    def body(x_ref, ss_ref, out_ref):
        out_ref[...] = jnp.maximum(
            x_ref[...] * ss_ref[0:1, :] + ss_ref[1:2, :], 0.0)

    return pl.pallas_call(
        body,
        grid=(nb,),
        in_specs=[pl.BlockSpec((blk, c), lambda i: (i, 0)),
                  pl.BlockSpec(ss.shape, lambda i: (0, 0))],
        out_specs=pl.BlockSpec((blk, c), lambda i: (i, 0)),
        out_shape=jax.ShapeDtypeStruct((n_valid, c), jnp.float32),
    )(x, ss)


def _scale_shift(st, g, b, n):
    mu = st[0] / n
    var = st[1] / n - mu * mu
    scale = g * lax.rsqrt(var + 1e-5)
    shift = b - mu * scale
    return jnp.concatenate(
        [scale[None], shift[None], jnp.zeros((6, scale.shape[0]), jnp.float32)], 0)


def kernel(edge_rep, cycle_rep, e2c_src, e2c_dst, cycle_id, c2e_src, c2e_dst,
           cyc_W1, cyc_g1, cyc_b1, cyc_W2, cyc_g2, cyc_b2,
           edg_W1, edg_g1, edg_b1, edg_W2, edg_g2, edg_b2):
    E, _ = edge_rep.shape
    C_ROWS, _ = cycle_rep.shape
    P1 = e2c_src.shape[0]
    P2 = c2e_src.shape[0]
    N_CYC = 30000

    zeros_h = jnp.zeros((GB, HID), jnp.float32)

    def rup(n, m):
        return (n + m - 1) // m * m

    p1p = rup(P1, NTILE * IC)
    e2c = _make_scatter(p1p, C_ROWS)(
        edge_rep,
        _pad1(e2c_src, p1p, 0),
        _pad1(e2c_dst, p1p, BIG),
        zeros_h)

    cp = rup(C_ROWS, NTILE * IC)
    cyc_sum = _make_scatter(cp, N_CYC)(
        e2c,
        _pad1(jnp.arange(C_ROWS, dtype=jnp.int32), cp, 0),
        _pad1(cycle_id.astype(jnp.int32), cp, BIG),
        zeros_h)

    gp = rup(C_ROWS, NSC * NTILE * GB)
    cyc_gath = _make_gather(gp)(
        cyc_sum, _pad1(cycle_id.astype(jnp.int32), gp, 0))

    h1, st1 = _mm_stats([cycle_rep, cyc_gath, e2c], cyc_W1, C_ROWS)
    ss1 = _scale_shift(st1, cyc_g1, cyc_b1, C_ROWS)
    h2, st2 = _bn_mm_stats(h1, ss1, cyc_W2, C_ROWS)
    ss2 = _scale_shift(st2, cyc_g2, cyc_b2, C_ROWS)
    cycle_out = _bn_apply(h2, ss2, C_ROWS)

    p2p = rup(P2, NTILE * IC)
    c2e = _make_scatter(p2p, E)(
        cycle_out,
        _pad1(c2e_src, p2p, 0),
        _pad1(c2e_dst, p2p, BIG),
        zeros_h)

    g1h, st3 = _mm_stats([edge_rep, c2e], edg_W1, E)
    ss3 = _scale_shift(st3, edg_g1, edg_b1, E)
    g2h, st4 = _bn_mm_stats(g1h, ss3, edg_W2, E)
    ss4 = _scale_shift(st4, edg_g2, edg_b2, E)
    edge_out = _bn_apply(g2h, ss4, E)

    return (edge_out, cycle_out)


# speedup vs baseline: 1.1911x; 1.0033x over previous
import functools

import jax
import jax.numpy as jnp
from jax import lax
from jax.experimental import pallas as pl
from jax.experimental.pallas import tpu as pltpu
from jax.experimental.pallas import tpu_sc as plsc

HID = 128
NSC = 2
NTILE = 16
CHUNK = 10112
NDUMP = 8
GB = 128
BIG = 1 << 30


def _pad1(x, n, val):
    return jnp.pad(x, (0, n - x.shape[0]), constant_values=val)


IC = 3840


def _make_scatter(n_pairs_pad, n_dest):
    assert n_pairs_pad % (NTILE * IC) == 0
    ppt = n_pairs_pad // NTILE
    nchunks = ppt // IC
    npass = -(-n_dest // (NSC * CHUNK))
    n_out_pad = npass * NSC * CHUNK
    zrows = CHUNK // NTILE
    mesh = plsc.VectorSubcoreMesh(core_axis_name="c", subcore_axis_name="s")

    @functools.partial(
        pl.kernel,
        out_type=jax.ShapeDtypeStruct((n_out_pad, HID), jnp.float32),
        mesh=mesh,
        scratch_types=[
            pltpu.VMEM((IC,), jnp.int32),
            pltpu.VMEM((IC,), jnp.int32),
            pltpu.VMEM((IC + 2 * GB,), jnp.int32),
            pltpu.VMEM((IC + 2 * GB,), jnp.int32),
            pltpu.VMEM((GB,), jnp.int32),
            pltpu.VMEM((GB, HID), jnp.float32),
            pltpu.VMEM((GB, HID), jnp.float32),
            pltpu.VMEM_SHARED((CHUNK + NDUMP, HID), jnp.float32),
            pltpu.SemaphoreType.DMA,
            pltpu.SemaphoreType.DMA,
        ],
        compiler_params=pltpu.CompilerParams(needs_layout_passes=False),
    )
    def k(table, srcs, dsts, zeros_h, out,
          srcv, dstv, msrc, mdst, idxb, rows0, rows1, shared, sem0, sem1):
        c = lax.axis_index("c")
        s = lax.axis_index("s")

        def one_pass(p, _):
            base = (p * NSC + c) * CHUNK
            for z in range(zrows // GB):
                pltpu.sync_copy(zeros_h,
                                shared.at[pl.ds(s * zrows + z * GB, GB)])
            if zrows % GB:
                rem = zrows % GB
                pltpu.sync_copy(
                    zeros_h.at[pl.ds(0, rem)],
                    shared.at[pl.ds(s * zrows + (zrows // GB) * GB, rem)])
            plsc.subcore_barrier()

            def chunk_body(kk, _):
                pltpu.sync_copy(srcs.at[pl.ds(s * ppt + kk * IC, IC)], srcv)
                pltpu.sync_copy(dsts.at[pl.ds(s * ppt + kk * IC, IC)], dstv)

                def scan(i, cnt):
                    dv = dstv[pl.ds(i * 16, 16)]
                    sv = srcv[pl.ds(i * 16, 16)]
                    lo = dv - base
                    m = (lo >= 0) & (lo < CHUNK)
                    mi = jnp.where(m, 1, 0)
                    csum = plsc.cumsum(mi)
                    pos = (cnt - 1) + csum
                    plsc.store_scatter(msrc, [pos], sv, mask=m)
                    plsc.store_scatter(mdst, [pos], lo, mask=m)
                    return cnt + jnp.squeeze(lax.slice(csum, (15,), (16,)))

                cnt = lax.fori_loop(0, IC // 16, scan, jnp.int32(0),
                                    unroll=4)
                for t in range(GB // 16):
                    lane = lax.iota(jnp.int32, 16)
                    msrc[pl.ds(cnt + t * 16, 16)] = (lane + s * 16) % 64
                    mdst[pl.ds(cnt + t * 16, 16)] = CHUNK + (lane + t) % NDUMP
                nb = (cnt + GB - 1) // GB

                @pl.when(nb > 0)
                def _():
                    pltpu.async_copy(
                        table.at[msrc.at[pl.ds(0, GB)]], rows0, sem0)

                def proc2(j2, _):
                    for par in range(2):
                        rbuf, rsem = (rows0, sem0) if par == 0 else (rows1, sem1)
                        obuf, osem = (rows1, sem1) if par == 0 else (rows0, sem0)
                        j = j2 * 2 + par

                        @pl.when(j < nb)
                        def _(j=j, rbuf=rbuf, rsem=rsem, obuf=obuf, osem=osem):
                            pltpu.make_async_copy(
                                table.at[msrc.at[pl.ds(j * GB, GB)]],
                                rbuf, rsem).wait()

                            @pl.when(j + 1 < nb)
                            def _():
                                pltpu.async_copy(
                                    table.at[msrc.at[pl.ds((j + 1) * GB, GB)]],
                                    obuf, osem)

                            for q in range(GB // 16):
                                idxb[pl.ds(q * 16, 16)] = (
                                    mdst[pl.ds(j * GB + q * 16, 16)])
                            pltpu.sync_copy(rbuf, shared.at[idxb], add=True)
                    return 0

                lax.fori_loop(0, (nb + 1) // 2, proc2, 0)
                return 0

            lax.fori_loop(0, nchunks, chunk_body, 0)
            plsc.subcore_barrier()
            pltpu.sync_copy(shared.at[pl.ds(s * zrows, zrows)],
                            out.at[pl.ds(base + s * zrows, zrows)])
            return 0

        lax.fori_loop(0, npass, one_pass, 0)

    return k


def _make_gather(n_rows_pad):
    assert n_rows_pad % (NSC * NTILE * GB) == 0
    per_w = n_rows_pad // (NSC * NTILE)
    nb = per_w // GB
    mesh = plsc.VectorSubcoreMesh(core_axis_name="c", subcore_axis_name="s")

    @functools.partial(
        pl.kernel,
        out_type=jax.ShapeDtypeStruct((n_rows_pad, HID), jnp.float32),
        mesh=mesh,
        scratch_types=[
            pltpu.VMEM((per_w,), jnp.int32),
            pltpu.VMEM((GB, HID), jnp.float32),
            pltpu.VMEM((GB, HID), jnp.float32),
            pltpu.SemaphoreType.DMA,
            pltpu.SemaphoreType.DMA,
        ],
        compiler_params=pltpu.CompilerParams(needs_layout_passes=False),
    )
    def k(table, idx, out, idxv, rows0, rows1, sem0, sem1):
        c = lax.axis_index("c")
        s = lax.axis_index("s")
        base = (s * NSC + c) * per_w
        pltpu.sync_copy(idx.at[pl.ds(base, per_w)], idxv)
        pltpu.async_copy(table.at[idxv.at[pl.ds(0, GB)]], rows0, sem0)

        def body(j2, _):
            for par in range(2):
                rbuf, rsem = (rows0, sem0) if par == 0 else (rows1, sem1)
                obuf, osem = (rows1, sem1) if par == 0 else (rows0, sem0)
                j = j2 * 2 + par

                @pl.when(j < nb)
                def _(j=j, rbuf=rbuf, rsem=rsem, obuf=obuf, osem=osem):
                    pltpu.make_async_copy(
                        table.at[idxv.at[pl.ds(j * GB, GB)]], rbuf, rsem
                    ).wait()

                    @pl.when(j + 1 < nb)
                    def _():
                        pltpu.async_copy(
                            table.at[idxv.at[pl.ds((j + 1) * GB, GB)]],
                            obuf, osem)

                    pltpu.sync_copy(rbuf, out.at[pl.ds(base + j * GB, GB)])
            return 0

        lax.fori_loop(0, (nb + 1) // 2, body, 0)

    return k


_BLK = 8192


def _mm_plain(x, w, n_valid):
    cout = w.shape[1]
    nb = pl.cdiv(n_valid, _BLK)

    def body(x_ref, w_ref, out_ref):
        out_ref[...] = jnp.dot(x_ref[...], w_ref[...],
                               preferred_element_type=jnp.float32)

    return pl.pallas_call(
        body,
        grid=(nb,),
        in_specs=[pl.BlockSpec((_BLK, x.shape[1]), lambda i: (i, 0)),
                  pl.BlockSpec(w.shape, lambda i: (0, 0))],
        out_specs=pl.BlockSpec((_BLK, cout), lambda i: (i, 0)),
        out_shape=jax.ShapeDtypeStruct((nb * _BLK, cout), jnp.float32),
    )(x, w)


def _mm_stats(parts, w, n_valid, pre=None):
    npart = len(parts)
    cout = w.shape[1]
    nb = pl.cdiv(n_valid, _BLK)
    has_pre = pre is not None
    ins = list(parts) + ([pre] if has_pre else [])

    def body(*refs):
        part_refs = refs[:npart]
        pre_ref = refs[npart] if has_pre else None
        w_ref, out_ref, st_ref = refs[npart + has_pre:]
        i = pl.program_id(0)
        rowid = lax.broadcasted_iota(jnp.int32, (_BLK, 1), 0) + i * _BLK
        valid = rowid < n_valid
        h = pre_ref[...] if has_pre else jnp.zeros((_BLK, cout), jnp.float32)
        for kk in range(npart):
            h = h + jnp.dot(part_refs[kk][...].astype(jnp.bfloat16),
                            w_ref[kk * HID:(kk + 1) * HID, :].astype(jnp.bfloat16),
                            preferred_element_type=jnp.float32)
        h = jnp.where(valid, h, 0.0)
        out_ref[...] = h
        su = jnp.sum(h, axis=0, keepdims=True)
        sq = jnp.sum(h * h, axis=0, keepdims=True)
        upd = jnp.concatenate([su, sq, jnp.zeros((6, cout), jnp.float32)], 0)

        @pl.when(i == 0)
        def _():
            st_ref[...] = upd

        @pl.when(i > 0)
        def _():
            st_ref[...] += upd

    out, st = pl.pallas_call(
        body,
        grid=(nb,),
        in_specs=[pl.BlockSpec((_BLK, HID), lambda i: (i, 0))] * npart
        + ([pl.BlockSpec((_BLK, cout), lambda i: (i, 0))] if has_pre else [])
        + [pl.BlockSpec(w.shape, lambda i: (0, 0))],
        out_specs=[pl.BlockSpec((_BLK, cout), lambda i: (i, 0)),
                   pl.BlockSpec((8, cout), lambda i: (0, 0))],
        out_shape=[jax.ShapeDtypeStruct((nb * _BLK, cout), jnp.float32),
                   jax.ShapeDtypeStruct((8, cout), jnp.float32)],
    )(*ins, w)
    return out, st


def _bn_mm_stats(x, ss, w, n_valid):
    cout = w.shape[1]
    nb = x.shape[0] // _BLK

    def body(x_ref, ss_ref, w_ref, out_ref, st_ref):
        i = pl.program_id(0)
        rowid = lax.broadcasted_iota(jnp.int32, (_BLK, 1), 0) + i * _BLK
        valid = rowid < n_valid
        a = jnp.maximum(x_ref[...] * ss_ref[0:1, :] + ss_ref[1:2, :], 0.0)
        a = jnp.where(valid, a, 0.0)
        h = jnp.dot(a.astype(jnp.bfloat16), w_ref[...].astype(jnp.bfloat16),
                    preferred_element_type=jnp.float32)
        out_ref[...] = h
        su = jnp.sum(h, axis=0, keepdims=True)
        sq = jnp.sum(h * h, axis=0, keepdims=True)
        upd = jnp.concatenate([su, sq, jnp.zeros((6, cout), jnp.float32)], 0)

        @pl.when(i == 0)
        def _():
            st_ref[...] = upd

        @pl.when(i > 0)
        def _():
            st_ref[...] += upd

    return pl.pallas_call(
        body,
        grid=(nb,),
        in_specs=[pl.BlockSpec((_BLK, x.shape[1]), lambda i: (i, 0)),
                  pl.BlockSpec(ss.shape, lambda i: (0, 0)),
                  pl.BlockSpec(w.shape, lambda i: (0, 0))],
        out_specs=[pl.BlockSpec((_BLK, cout), lambda i: (i, 0)),
                   pl.BlockSpec((8, cout), lambda i: (0, 0))],
        out_shape=[jax.ShapeDtypeStruct((nb * _BLK, cout), jnp.float32),
                   jax.ShapeDtypeStruct((8, cout), jnp.float32)],
    )(x, ss, w)


def _bn_apply(x, ss, n_valid):
    c = x.shape[1]
    blk = 1024
    nb = pl.cdiv(n_valid, blk)

    def body(x_ref, ss_ref, out_ref):
        out_ref[...] = jnp.maximum(
            x_ref[...] * ss_ref[0:1, :] + ss_ref[1:2, :], 0.0)

    return pl.pallas_call(
        body,
        grid=(nb,),
        in_specs=[pl.BlockSpec((blk, c), lambda i: (i, 0)),
                  pl.BlockSpec(ss.shape, lambda i: (0, 0))],
        out_specs=pl.BlockSpec((blk, c), lambda i: (i, 0)),
        out_shape=jax.ShapeDtypeStruct((n_valid, c), jnp.float32),
    )(x, ss)


def _scale_shift(st, g, b, n):
    mu = st[0] / n
    var = st[1] / n - mu * mu
    scale = g * lax.rsqrt(var + 1e-5)
    shift = b - mu * scale
    return jnp.concatenate(
        [scale[None], shift[None], jnp.zeros((6, scale.shape[0]), jnp.float32)], 0)


def kernel(edge_rep, cycle_rep, e2c_src, e2c_dst, cycle_id, c2e_src, c2e_dst,
           cyc_W1, cyc_g1, cyc_b1, cyc_W2, cyc_g2, cyc_b2,
           edg_W1, edg_g1, edg_b1, edg_W2, edg_g2, edg_b2):
    E, _ = edge_rep.shape
    C_ROWS, _ = cycle_rep.shape
    P1 = e2c_src.shape[0]
    P2 = c2e_src.shape[0]
    N_CYC = 30000

    zeros_h = jnp.zeros((GB, HID), jnp.float32)

    def rup(n, m):
        return (n + m - 1) // m * m

    p1p = rup(P1, NTILE * IC)
    e2c = _make_scatter(p1p, C_ROWS)(
        edge_rep,
        _pad1(e2c_src, p1p, 0),
        _pad1(e2c_dst, p1p, BIG),
        zeros_h)

    cp = rup(C_ROWS, NTILE * IC)
    cyc_sum = _make_scatter(cp, N_CYC)(
        e2c,
        _pad1(jnp.arange(C_ROWS, dtype=jnp.int32), cp, 0),
        _pad1(cycle_id.astype(jnp.int32), cp, BIG),
        zeros_h)

    gp = rup(C_ROWS, NSC * NTILE * GB)
    cyc_gath = _make_gather(gp)(
        cyc_sum, _pad1(cycle_id.astype(jnp.int32), gp, 0))

    h1, st1 = _mm_stats([cycle_rep, cyc_gath, e2c], cyc_W1, C_ROWS)
    ss1 = _scale_shift(st1, cyc_g1, cyc_b1, C_ROWS)
    h2, st2 = _bn_mm_stats(h1, ss1, cyc_W2, C_ROWS)
    ss2 = _scale_shift(st2, cyc_g2, cyc_b2, C_ROWS)
    cycle_out = _bn_apply(h2, ss2, C_ROWS)

    p2p = rup(P2, NTILE * IC)
    c2e = _make_scatter(p2p, E)(
        cycle_out,
        _pad1(c2e_src, p2p, 0),
        _pad1(c2e_dst, p2p, BIG),
        zeros_h)

    g1h, st3 = _mm_stats([edge_rep, c2e], edg_W1, E)
    ss3 = _scale_shift(st3, edg_g1, edg_b1, E)
    g2h, st4 = _bn_mm_stats(g1h, ss3, edg_W2, E)
    ss4 = _scale_shift(st4, edg_g2, edg_b2, E)
    edge_out = _bn_apply(g2h, ss4, E)

    return (edge_out, cycle_out)
